# baseline (speedup 1.0000x reference)
import functools

import jax
import jax.numpy as jnp
from jax import lax
from jax.experimental import pallas as pl
from jax.experimental.pallas import tpu as pltpu
from jax.experimental.pallas import tpu_sc as plsc

N = 10000
E = 160000
ETOT = E + N
NC = 2
NS = 16
NW = NC * NS
LANES = 16
GEO1 = (128, 56, 28)
GEO2 = (112, 64, 32)
NPAD = 10112
STRIPE = NPAD // NS

C1, CP1 = 66, 80
C2, CP2 = 100, 112
B = 50
NODES_PER_G = 200
OUT_DIM = 200

_SELU_ALPHA = 1.6732632423543772
_SELU_SCALE = 1.0507009873554805


def _selu(v):
    neg = _SELU_ALPHA * (jnp.exp(jnp.minimum(v, 0.0)) - 1.0)
    return _SELU_SCALE * jnp.where(v > 0, v, neg)


@functools.lru_cache(maxsize=None)
def _edge_pass(cp, geo):
    chunk, nch_a, nch_b = geo
    pt_a = nch_a * chunk
    pt_b = nch_b * chunk
    off_b = NS * pt_a
    mesh = plsc.VectorSubcoreMesh(
        core_axis_name="c", subcore_axis_name="s", num_cores=NC, num_subcores=NS)

    @functools.partial(
        pl.kernel,
        out_type=jax.ShapeDtypeStruct((NC, NPAD, cp), jnp.float32),
        mesh=mesh,
        scratch_types=[
            pltpu.VMEM((nch_a, chunk), jnp.int32),
            pltpu.VMEM((nch_a, chunk), jnp.int32),
            pltpu.VMEM((2, N), jnp.float32),
            pltpu.VMEM((chunk,), jnp.float32),
            pltpu.VMEM((chunk, cp), jnp.float32),
            pltpu.VMEM((chunk, cp), jnp.float32),
            pltpu.VMEM_SHARED((NPAD, cp), jnp.float32),
            pltpu.SemaphoreType.DMA,
            pltpu.SemaphoreType.DMA,
        ],
        compiler_params=pltpu.CompilerParams(
            needs_layout_passes=False, use_tc_tiling_on_sc=False),
    )
    def edge_pass(h_hbm, src_hbm, dst_hbm, aa_hbm, z_hbm, out_hbm,
                  src_v, dst_v, aa_v, ex_v, rows0_v, rows1_v, acc_sh,
                  sem0, sem1):
        cid = lax.axis_index("c")
        sid = lax.axis_index("s")
        wid = cid * NS + sid
        pltpu.sync_copy(aa_hbm, aa_v)
        pltpu.sync_copy(src_hbm.at[wid], src_v)
        pltpu.sync_copy(dst_hbm.at[wid], dst_v)
        pltpu.sync_copy(z_hbm.at[pl.ds(sid * STRIPE, STRIPE)],
                        acc_sh.at[pl.ds(sid * STRIPE, STRIPE)])

        lanes = lax.iota(jnp.int32, LANES)
        zeros16 = jnp.zeros((LANES,), jnp.int32)
        ones16 = jnp.ones((LANES,), jnp.int32)

        def _mx(i, m):
            ma, md = m
            lin = i * LANES + lanes
            return (jnp.maximum(ma, plsc.load_gather(aa_v, [zeros16, lin])),
                    jnp.maximum(md, plsc.load_gather(aa_v, [ones16, lin])))
        ninf = jnp.full((LANES,), -jnp.inf, jnp.float32)
        ma, md = lax.fori_loop(0, N // LANES, _mx, (ninf, ninf))
        plsc.store_scatter(ex_v, [lanes], ma)
        plsc.store_scatter(ex_v, [LANES + lanes], md)
        ga, gd = ninf, ninf
        for k in range(LANES):
            ga = jnp.maximum(ga, plsc.load_gather(
                ex_v, [jnp.full((LANES,), k, jnp.int32)]))
            gd = jnp.maximum(gd, plsc.load_gather(
                ex_v, [jnp.full((LANES,), LANES + k, jnp.int32)]))
        g0 = ga + gd
        g = jnp.maximum(g0, 0.2 * g0)

        plsc.subcore_barrier()
        base = lax.select(cid == 1, sid * pt_a, off_b + sid * pt_b)
        nch = lax.select(cid == 1, nch_a, nch_b)

        def process(j, rows, sgem):
            j16 = jnp.full((LANES,), j, jnp.int32)
            for t in range(chunk // LANES):
                s16 = plsc.load_gather(src_v, [j16, t * LANES + lanes])
                d16 = plsc.load_gather(dst_v, [j16, t * LANES + lanes])
                a = (plsc.load_gather(aa_v, [zeros16, s16]) +
                     plsc.load_gather(aa_v, [ones16, d16]))
                a = jnp.maximum(a, 0.2 * a)
                ex = jnp.exp(a - g)
                eid = base + j * chunk + t * LANES + lanes
                plsc.store_scatter(ex_v, [t * LANES + lanes],
                                   jnp.where(eid < ETOT, ex, 0.0))
            pltpu.make_async_copy(h_hbm.at[src_v.at[j]], rows, sgem).wait()

            @plsc.parallel_loop(0, chunk, unroll=4)
            def scale_body(k):
                k16 = jnp.full((LANES,), k, jnp.int32)
                exb = plsc.load_gather(ex_v, [k16])
                for c in range(cp // LANES):
                    v = plsc.load_gather(rows, [k16, c * LANES + lanes])
                    plsc.store_scatter(rows, [k16, c * LANES + lanes], v * exb)

            pltpu.sync_copy(rows, acc_sh.at[dst_v.at[j]], add=True)

        pltpu.async_copy(h_hbm.at[src_v.at[0]], rows0_v, sem0)

        def outer(i, _):
            for b in range(2):
                j = 2 * i + b
                rows, sgem = (rows0_v, sem0) if b == 0 else (rows1_v, sem1)
                orows, osem = (rows1_v, sem1) if b == 0 else (rows0_v, sem0)

                @pl.when(j + 1 < nch)
                def _issue():
                    pltpu.async_copy(h_hbm.at[src_v.at[j + 1]], orows, osem)

                process(j, rows, sgem)
            return 0

        lax.fori_loop(0, nch // 2, outer, 0)
        plsc.subcore_barrier()
        pltpu.sync_copy(acc_sh.at[pl.ds(sid * STRIPE, STRIPE)],
                        out_hbm.at[cid, pl.ds(sid * STRIPE, STRIPE)])

    return edge_pass


def _tc_prep_body(x_ref, w_ref, asr_ref, adr_ref, h_ref, aa_ref):
    h = jnp.dot(x_ref[...], w_ref[...], preferred_element_type=jnp.float32)
    asv = jnp.sum(h * asr_ref[...], axis=1)
    adv = jnp.sum(h * adr_ref[...], axis=1)
    h_ref[...] = jnp.concatenate(
        [h, jnp.ones((N, 1), jnp.float32),
         jnp.zeros((N, CP1 - C1 - 1), jnp.float32)], axis=1)
    aa_ref[...] = jnp.stack([asv, adv])


def _tc_mid_body(p_ref, b1_ref, w2_ref, a2s_ref, a2d_ref, h_ref, aa_ref):
    p0 = p_ref[0, :N]
    p1 = p_ref[1, :N]
    num = p0[:, :C1] + p1[:, :C1]
    den = p0[:, C1:C1 + 1] + p1[:, C1:C1 + 1] + 1e-16
    o1 = _selu(num / den + b1_ref[...])
    h2 = jnp.dot(o1, w2_ref[...], preferred_element_type=jnp.float32)
    asv = jnp.sum(h2 * a2s_ref[...], axis=1)
    adv = jnp.sum(h2 * a2d_ref[...], axis=1)
    h_ref[...] = jnp.concatenate(
        [h2, jnp.ones((N, 1), jnp.float32),
         jnp.zeros((N, CP2 - C2 - 1), jnp.float32)], axis=1)
    aa_ref[...] = jnp.stack([asv, adv])


def _tc_fina_body(p_ref, b2_ref, o_ref):
    p0 = p_ref[0, :N]
    p1 = p_ref[1, :N]
    num = p0[:, :C2] + p1[:, :C2]
    den = p0[:, C2:C2 + 1] + p1[:, C2:C2 + 1] + 1e-16
    o_ref[...] = _selu(num / den + b2_ref[...])


def _tc_finb_body(o_ref, w1_ref, b1_ref, w2_ref, b2_ref, m_ref, z_ref):
    y = lax.dot_general(o_ref[...], w1_ref[...], (((1,), (1,)), ((), ())),
                        preferred_element_type=jnp.float32)
    y = _selu(y + b1_ref[...])
    z = lax.dot_general(y, w2_ref[...], (((1,), (1,)), ((), ())),
                        preferred_element_type=jnp.float32)
    z = z + b2_ref[...]
    z_ref[...] = jnp.where(m_ref[...], z, -jnp.inf)


def kernel(x, edge_index, mask, W1, att_src1, att_dst1, b1,
           W2, att_src2, att_dst2, b2, fc1_w, fc1_b, fc2_w, fc2_b):
    loop = jnp.arange(N, dtype=jnp.int32)

    def _slabs(e, geo):
        chunk, nch_a, nch_b = geo
        off_b = NS * nch_a * chunk
        total = NS * (nch_a + nch_b) * chunk
        flat = jnp.concatenate(
            [e.astype(jnp.int32), loop,
             jnp.zeros((total - ETOT,), jnp.int32)])
        a = flat[:off_b].reshape(NS, nch_a, chunk)
        bpart = flat[off_b:].reshape(NS, nch_b, chunk)
        padch = jnp.zeros((NS, nch_a - nch_b, chunk), jnp.int32)
        return jnp.concatenate([a, jnp.concatenate([bpart, padch], axis=1)])

    srcp1 = _slabs(edge_index[0], GEO1)
    dstp1 = _slabs(edge_index[1], GEO1)
    srcp2 = _slabs(edge_index[0], GEO2)
    dstp2 = _slabs(edge_index[1], GEO2)
    z1 = jnp.zeros((NPAD, CP1), jnp.float32)
    z2 = jnp.zeros((NPAD, CP2), jnp.float32)

    h1pad, aa1 = pl.pallas_call(
        _tc_prep_body,
        out_shape=(jax.ShapeDtypeStruct((N, CP1), jnp.float32),
                   jax.ShapeDtypeStruct((2, N), jnp.float32)),
    )(x, W1, att_src1.reshape(1, C1), att_dst1.reshape(1, C1))

    p1 = _edge_pass(CP1, GEO1)(h1pad, srcp1, dstp1, aa1, z1)

    h2pad, aa2 = pl.pallas_call(
        _tc_mid_body,
        out_shape=(jax.ShapeDtypeStruct((N, CP2), jnp.float32),
                   jax.ShapeDtypeStruct((2, N), jnp.float32)),
    )(p1, b1.reshape(1, C1), W2, att_src2.reshape(1, C2),
      att_dst2.reshape(1, C2))

    p2 = _edge_pass(CP2, GEO2)(h2pad, srcp2, dstp2, aa2, z2)

    o2 = pl.pallas_call(
        _tc_fina_body,
        out_shape=jax.ShapeDtypeStruct((N, C2), jnp.float32),
    )(p2, b2.reshape(1, C2))

    o2f = o2.reshape(B, NODES_PER_G * C2)

    out = pl.pallas_call(
        _tc_finb_body,
        out_shape=jax.ShapeDtypeStruct((B, OUT_DIM), jnp.float32),
    )(o2f, fc1_w, fc1_b.reshape(1, OUT_DIM), fc2_w,
      fc2_b.reshape(1, OUT_DIM), mask)
    return out


# speedup vs baseline: 9.2259x; 9.2259x over previous
import functools

import jax
import jax.numpy as jnp
from jax import lax
from jax.experimental import pallas as pl
from jax.experimental.pallas import tpu as pltpu
from jax.experimental.pallas import tpu_sc as plsc

N = 10000
E = 160000
ETOT = E + N
NC = 2
NS = 16
NW = NC * NS
LANES = 16
GEO1 = (128, 56, 28)
GEO2 = (112, 64, 32)
BIGCORE = 1
NPAD = 10112
STRIPE = NPAD // NS

C1, CP1 = 66, 80
C2, CP2 = 100, 112
B = 50
NODES_PER_G = 200
OUT_DIM = 200

_SELU_ALPHA = 1.6732632423543772
_SELU_SCALE = 1.0507009873554805


def _selu(v):
    neg = _SELU_ALPHA * (jnp.exp(jnp.minimum(v, 0.0)) - 1.0)
    return _SELU_SCALE * jnp.where(v > 0, v, neg)


@functools.lru_cache(maxsize=None)
def _edge_pass(cp, geo):
    chunk, nch_a, nch_b = geo
    pt_a = nch_a * chunk
    pt_b = nch_b * chunk
    off_b = NS * pt_a
    mesh = plsc.VectorSubcoreMesh(
        core_axis_name="c", subcore_axis_name="s", num_cores=NC, num_subcores=NS)

    @functools.partial(
        pl.kernel,
        out_type=jax.ShapeDtypeStruct((NC, NPAD, cp), jnp.float32),
        mesh=mesh,
        scratch_types=[
            pltpu.VMEM((nch_a, chunk), jnp.int32),
            pltpu.VMEM((nch_a, chunk), jnp.int32),
            pltpu.VMEM((2, N), jnp.float32),
            pltpu.VMEM((chunk,), jnp.float32),
            pltpu.VMEM((chunk, cp), jnp.float32),
            pltpu.VMEM((chunk, cp), jnp.float32),
            pltpu.VMEM_SHARED((NPAD, cp), jnp.float32),
            pltpu.SemaphoreType.DMA,
            pltpu.SemaphoreType.DMA,
        ],
        compiler_params=pltpu.CompilerParams(
            needs_layout_passes=False, use_tc_tiling_on_sc=False),
    )
    def edge_pass(h_hbm, src_hbm, dst_hbm, aa_hbm, z_hbm, out_hbm,
                  src_v, dst_v, aa_v, ex_v, rows0_v, rows1_v, acc_sh,
                  sem0, sem1):
        cid = lax.axis_index("c")
        sid = lax.axis_index("s")
        slab = lax.select(cid == BIGCORE, sid, NS + sid)
        pltpu.sync_copy(aa_hbm, aa_v)
        pltpu.sync_copy(src_hbm.at[slab], src_v)
        pltpu.sync_copy(dst_hbm.at[slab], dst_v)
        pltpu.sync_copy(z_hbm.at[pl.ds(sid * STRIPE, STRIPE)],
                        acc_sh.at[pl.ds(sid * STRIPE, STRIPE)])

        lanes = lax.iota(jnp.int32, LANES)
        zeros16 = jnp.zeros((LANES,), jnp.int32)
        ones16 = jnp.ones((LANES,), jnp.int32)

        def _mx(i, m):
            ma, md = m
            lin = i * LANES + lanes
            return (jnp.maximum(ma, plsc.load_gather(aa_v, [zeros16, lin])),
                    jnp.maximum(md, plsc.load_gather(aa_v, [ones16, lin])))
        ninf = jnp.full((LANES,), -jnp.inf, jnp.float32)
        ma, md = lax.fori_loop(0, N // LANES, _mx, (ninf, ninf))
        plsc.store_scatter(ex_v, [lanes], ma)
        plsc.store_scatter(ex_v, [LANES + lanes], md)
        ga, gd = ninf, ninf
        for k in range(LANES):
            ga = jnp.maximum(ga, plsc.load_gather(
                ex_v, [jnp.full((LANES,), k, jnp.int32)]))
            gd = jnp.maximum(gd, plsc.load_gather(
                ex_v, [jnp.full((LANES,), LANES + k, jnp.int32)]))
        g0 = ga + gd
        g = jnp.maximum(g0, 0.2 * g0)

        plsc.subcore_barrier()
        base = lax.select(cid == BIGCORE, sid * pt_a, off_b + sid * pt_b)
        nch = lax.select(cid == BIGCORE, nch_a, nch_b)

        def process(j, rows, sgem):
            j16 = jnp.full((LANES,), j, jnp.int32)
            for t in range(chunk // LANES):
                s16 = plsc.load_gather(src_v, [j16, t * LANES + lanes])
                d16 = plsc.load_gather(dst_v, [j16, t * LANES + lanes])
                a = (plsc.load_gather(aa_v, [zeros16, s16]) +
                     plsc.load_gather(aa_v, [ones16, d16]))
                a = jnp.maximum(a, 0.2 * a)
                ex = jnp.exp(a - g)
                eid = base + j * chunk + t * LANES + lanes
                plsc.store_scatter(ex_v, [t * LANES + lanes],
                                   jnp.where(eid < ETOT, ex, 0.0))
            pltpu.make_async_copy(h_hbm.at[src_v.at[j]], rows, sgem).wait()

            @plsc.parallel_loop(0, chunk, unroll=4)
            def scale_body(k):
                k16 = jnp.full((LANES,), k, jnp.int32)
                exb = plsc.load_gather(ex_v, [k16])
                for c in range(cp // LANES):
                    v = plsc.load_gather(rows, [k16, c * LANES + lanes])
                    plsc.store_scatter(rows, [k16, c * LANES + lanes], v * exb)

            pltpu.sync_copy(rows, acc_sh.at[dst_v.at[j]], add=True)

        pltpu.async_copy(h_hbm.at[src_v.at[0]], rows0_v, sem0)

        def outer(i, _):
            for b in range(2):
                j = 2 * i + b
                rows, sgem = (rows0_v, sem0) if b == 0 else (rows1_v, sem1)
                orows, osem = (rows1_v, sem1) if b == 0 else (rows0_v, sem0)

                @pl.when(j + 1 < nch)
                def _issue():
                    pltpu.async_copy(h_hbm.at[src_v.at[j + 1]], orows, osem)

                process(j, rows, sgem)
            return 0

        lax.fori_loop(0, nch // 2, outer, 0)
        plsc.subcore_barrier()
        pltpu.sync_copy(acc_sh.at[pl.ds(sid * STRIPE, STRIPE)],
                        out_hbm.at[cid, pl.ds(sid * STRIPE, STRIPE)])

    return edge_pass


def _tc_prep_body(x_ref, w_ref, asr_ref, adr_ref, h_ref, aa_ref):
    h = jnp.dot(x_ref[...], w_ref[...], preferred_element_type=jnp.float32)
    asv = jnp.sum(h * asr_ref[...], axis=1)
    adv = jnp.sum(h * adr_ref[...], axis=1)
    h_ref[...] = jnp.concatenate(
        [h, jnp.ones((N, 1), jnp.float32),
         jnp.zeros((N, CP1 - C1 - 1), jnp.float32)], axis=1)
    aa_ref[...] = jnp.stack([asv, adv])


def _tc_mid_body(p_ref, b1_ref, w2_ref, a2s_ref, a2d_ref, h_ref, aa_ref):
    p0 = p_ref[0, :N]
    p1 = p_ref[1, :N]
    num = p0[:, :C1] + p1[:, :C1]
    den = p0[:, C1:C1 + 1] + p1[:, C1:C1 + 1] + 1e-16
    o1 = _selu(num / den + b1_ref[...])
    h2 = jnp.dot(o1, w2_ref[...], preferred_element_type=jnp.float32)
    asv = jnp.sum(h2 * a2s_ref[...], axis=1)
    adv = jnp.sum(h2 * a2d_ref[...], axis=1)
    h_ref[...] = jnp.concatenate(
        [h2, jnp.ones((N, 1), jnp.float32),
         jnp.zeros((N, CP2 - C2 - 1), jnp.float32)], axis=1)
    aa_ref[...] = jnp.stack([asv, adv])


def _tc_fina_body(p_ref, b2_ref, o_ref):
    p0 = p_ref[0, :N]
    p1 = p_ref[1, :N]
    num = p0[:, :C2] + p1[:, :C2]
    den = p0[:, C2:C2 + 1] + p1[:, C2:C2 + 1] + 1e-16
    o_ref[...] = _selu(num / den + b2_ref[...])


def _tc_finb_body(o_ref, w1_ref, b1_ref, w2_ref, b2_ref, m_ref, z_ref):
    y = lax.dot_general(o_ref[...], w1_ref[...], (((1,), (1,)), ((), ())),
                        preferred_element_type=jnp.float32)
    y = _selu(y + b1_ref[...])
    z = lax.dot_general(y, w2_ref[...], (((1,), (1,)), ((), ())),
                        preferred_element_type=jnp.float32)
    z = z + b2_ref[...]
    z_ref[...] = jnp.where(m_ref[...], z, -jnp.inf)


def kernel(x, edge_index, mask, W1, att_src1, att_dst1, b1,
           W2, att_src2, att_dst2, b2, fc1_w, fc1_b, fc2_w, fc2_b):
    loop = jnp.arange(N, dtype=jnp.int32)

    def _slabs(e, geo):
        chunk, nch_a, nch_b = geo
        off_b = NS * nch_a * chunk
        total = NS * (nch_a + nch_b) * chunk
        flat = jnp.concatenate(
            [e.astype(jnp.int32), loop,
             jnp.zeros((total - ETOT,), jnp.int32)])
        a = flat[:off_b].reshape(NS, nch_a, chunk)
        bpart = flat[off_b:].reshape(NS, nch_b, chunk)
        padch = jnp.zeros((NS, nch_a - nch_b, chunk), jnp.int32)
        return jnp.concatenate([a, jnp.concatenate([bpart, padch], axis=1)])

    srcp1 = _slabs(edge_index[0], GEO1)
    dstp1 = _slabs(edge_index[1], GEO1)
    srcp2 = _slabs(edge_index[0], GEO2)
    dstp2 = _slabs(edge_index[1], GEO2)
    z1 = jnp.zeros((NPAD, CP1), jnp.float32)
    z2 = jnp.zeros((NPAD, CP2), jnp.float32)

    h1pad, aa1 = pl.pallas_call(
        _tc_prep_body,
        out_shape=(jax.ShapeDtypeStruct((N, CP1), jnp.float32),
                   jax.ShapeDtypeStruct((2, N), jnp.float32)),
    )(x, W1, att_src1.reshape(1, C1), att_dst1.reshape(1, C1))

    p1 = _edge_pass(CP1, GEO1)(h1pad, srcp1, dstp1, aa1, z1)

    h2pad, aa2 = pl.pallas_call(
        _tc_mid_body,
        out_shape=(jax.ShapeDtypeStruct((N, CP2), jnp.float32),
                   jax.ShapeDtypeStruct((2, N), jnp.float32)),
    )(p1, b1.reshape(1, C1), W2, att_src2.reshape(1, C2),
      att_dst2.reshape(1, C2))

    p2 = _edge_pass(CP2, GEO2)(h2pad, srcp2, dstp2, aa2, z2)

    o2 = pl.pallas_call(
        _tc_fina_body,
        out_shape=jax.ShapeDtypeStruct((N, C2), jnp.float32),
    )(p2, b2.reshape(1, C2))

    o2f = o2.reshape(B, NODES_PER_G * C2)

    out = pl.pallas_call(
        _tc_finb_body,
        out_shape=jax.ShapeDtypeStruct((B, OUT_DIM), jnp.float32),
    )(o2f, fc1_w, fc1_b.reshape(1, OUT_DIM), fc2_w,
      fc2_b.reshape(1, OUT_DIM), mask)
    return out


# speedup vs baseline: 9.7115x; 1.0526x over previous
import functools

import jax
import jax.numpy as jnp
from jax import lax
from jax.experimental import pallas as pl
from jax.experimental.pallas import tpu as pltpu
from jax.experimental.pallas import tpu_sc as plsc

N = 10000
E = 160000
ETOT = E + N
NC = 2
NS = 16
NW = NC * NS
LANES = 16
GEO1 = (112, 64, 32)
GEO2 = (112, 64, 32)
BIGCORE = 0
NPAD = 10112
STRIPE = NPAD // NS

C1, CP1 = 66, 80
C2, CP2 = 100, 112
B = 50
NODES_PER_G = 200
OUT_DIM = 200

_SELU_ALPHA = 1.6732632423543772
_SELU_SCALE = 1.0507009873554805


def _selu(v):
    neg = _SELU_ALPHA * (jnp.exp(jnp.minimum(v, 0.0)) - 1.0)
    return _SELU_SCALE * jnp.where(v > 0, v, neg)


@functools.lru_cache(maxsize=None)
def _edge_pass(cp, geo):
    chunk, nch_a, nch_b = geo
    pt_a = nch_a * chunk
    pt_b = nch_b * chunk
    off_b = NS * pt_a
    mesh = plsc.VectorSubcoreMesh(
        core_axis_name="c", subcore_axis_name="s", num_cores=NC, num_subcores=NS)

    @functools.partial(
        pl.kernel,
        out_type=jax.ShapeDtypeStruct((NC, NPAD, cp), jnp.float32),
        mesh=mesh,
        scratch_types=[
            pltpu.VMEM((nch_a, chunk), jnp.int32),
            pltpu.VMEM((nch_a, chunk), jnp.int32),
            pltpu.VMEM((2, N), jnp.float32),
            pltpu.VMEM((chunk,), jnp.float32),
            pltpu.VMEM((chunk, cp), jnp.float32),
            pltpu.VMEM((chunk, cp), jnp.float32),
            pltpu.VMEM_SHARED((NPAD, cp), jnp.float32),
            pltpu.SemaphoreType.DMA,
            pltpu.SemaphoreType.DMA,
        ],
        compiler_params=pltpu.CompilerParams(
            needs_layout_passes=False, use_tc_tiling_on_sc=False),
    )
    def edge_pass(h_hbm, src_hbm, dst_hbm, aa_hbm, z_hbm, out_hbm,
                  src_v, dst_v, aa_v, ex_v, rows0_v, rows1_v, acc_sh,
                  sem0, sem1):
        cid = lax.axis_index("c")
        sid = lax.axis_index("s")
        slab = lax.select(cid == BIGCORE, sid, NS + sid)
        pltpu.sync_copy(aa_hbm, aa_v)
        pltpu.sync_copy(src_hbm.at[slab], src_v)
        pltpu.sync_copy(dst_hbm.at[slab], dst_v)
        pltpu.sync_copy(z_hbm.at[pl.ds(sid * STRIPE, STRIPE)],
                        acc_sh.at[pl.ds(sid * STRIPE, STRIPE)])

        lanes = lax.iota(jnp.int32, LANES)
        zeros16 = jnp.zeros((LANES,), jnp.int32)
        ones16 = jnp.ones((LANES,), jnp.int32)

        def _mx(i, m):
            ma, md = m
            lin = i * LANES + lanes
            return (jnp.maximum(ma, plsc.load_gather(aa_v, [zeros16, lin])),
                    jnp.maximum(md, plsc.load_gather(aa_v, [ones16, lin])))
        ninf = jnp.full((LANES,), -jnp.inf, jnp.float32)
        ma, md = lax.fori_loop(0, N // LANES, _mx, (ninf, ninf))
        plsc.store_scatter(ex_v, [lanes], ma)
        plsc.store_scatter(ex_v, [LANES + lanes], md)
        ga, gd = ninf, ninf
        for k in range(LANES):
            ga = jnp.maximum(ga, plsc.load_gather(
                ex_v, [jnp.full((LANES,), k, jnp.int32)]))
            gd = jnp.maximum(gd, plsc.load_gather(
                ex_v, [jnp.full((LANES,), LANES + k, jnp.int32)]))
        g0 = ga + gd
        g = jnp.maximum(g0, 0.2 * g0)

        plsc.subcore_barrier()
        base = lax.select(cid == BIGCORE, sid * pt_a, off_b + sid * pt_b)
        nch = lax.select(cid == BIGCORE, nch_a, nch_b)

        def process(j, rows, sgem):
            j16 = jnp.full((LANES,), j, jnp.int32)
            for t in range(chunk // LANES):
                s16 = plsc.load_gather(src_v, [j16, t * LANES + lanes])
                d16 = plsc.load_gather(dst_v, [j16, t * LANES + lanes])
                a = (plsc.load_gather(aa_v, [zeros16, s16]) +
                     plsc.load_gather(aa_v, [ones16, d16]))
                a = jnp.maximum(a, 0.2 * a)
                ex = jnp.exp(a - g)
                eid = base + j * chunk + t * LANES + lanes
                plsc.store_scatter(ex_v, [t * LANES + lanes],
                                   jnp.where(eid < ETOT, ex, 0.0))
            pltpu.make_async_copy(h_hbm.at[src_v.at[j]], rows, sgem).wait()

            @plsc.parallel_loop(0, chunk, unroll=4)
            def scale_body(k):
                k16 = jnp.full((LANES,), k, jnp.int32)
                exb = plsc.load_gather(ex_v, [k16])
                for c in range(cp // LANES):
                    v = plsc.load_gather(rows, [k16, c * LANES + lanes])
                    plsc.store_scatter(rows, [k16, c * LANES + lanes], v * exb)

            pltpu.sync_copy(rows, acc_sh.at[dst_v.at[j]], add=True)

        pltpu.async_copy(h_hbm.at[src_v.at[0]], rows0_v, sem0)

        def outer(i, _):
            for b in range(2):
                j = 2 * i + b
                rows, sgem = (rows0_v, sem0) if b == 0 else (rows1_v, sem1)
                orows, osem = (rows1_v, sem1) if b == 0 else (rows0_v, sem0)

                @pl.when(j + 1 < nch)
                def _issue():
                    pltpu.async_copy(h_hbm.at[src_v.at[j + 1]], orows, osem)

                process(j, rows, sgem)
            return 0

        lax.fori_loop(0, nch // 2, outer, 0)
        plsc.subcore_barrier()
        pltpu.sync_copy(acc_sh.at[pl.ds(sid * STRIPE, STRIPE)],
                        out_hbm.at[cid, pl.ds(sid * STRIPE, STRIPE)])

    return edge_pass


def _tc_prep_body(x_ref, w_ref, asr_ref, adr_ref, h_ref, aa_ref):
    h = jnp.dot(x_ref[...], w_ref[...], preferred_element_type=jnp.float32)
    asv = jnp.sum(h * asr_ref[...], axis=1)
    adv = jnp.sum(h * adr_ref[...], axis=1)
    h_ref[...] = jnp.concatenate(
        [h, jnp.ones((N, 1), jnp.float32),
         jnp.zeros((N, CP1 - C1 - 1), jnp.float32)], axis=1)
    aa_ref[...] = jnp.stack([asv, adv])


def _tc_mid_body(p_ref, b1_ref, w2_ref, a2s_ref, a2d_ref, h_ref, aa_ref):
    p0 = p_ref[0, :N]
    p1 = p_ref[1, :N]
    num = p0[:, :C1] + p1[:, :C1]
    den = p0[:, C1:C1 + 1] + p1[:, C1:C1 + 1] + 1e-16
    o1 = _selu(num / den + b1_ref[...])
    h2 = jnp.dot(o1, w2_ref[...], preferred_element_type=jnp.float32)
    asv = jnp.sum(h2 * a2s_ref[...], axis=1)
    adv = jnp.sum(h2 * a2d_ref[...], axis=1)
    h_ref[...] = jnp.concatenate(
        [h2, jnp.ones((N, 1), jnp.float32),
         jnp.zeros((N, CP2 - C2 - 1), jnp.float32)], axis=1)
    aa_ref[...] = jnp.stack([asv, adv])


def _tc_fina_body(p_ref, b2_ref, o_ref):
    p0 = p_ref[0, :N]
    p1 = p_ref[1, :N]
    num = p0[:, :C2] + p1[:, :C2]
    den = p0[:, C2:C2 + 1] + p1[:, C2:C2 + 1] + 1e-16
    o_ref[...] = _selu(num / den + b2_ref[...])


def _tc_finb_body(o_ref, w1_ref, b1_ref, w2_ref, b2_ref, m_ref, z_ref):
    y = lax.dot_general(o_ref[...], w1_ref[...], (((1,), (1,)), ((), ())),
                        preferred_element_type=jnp.float32)
    y = _selu(y + b1_ref[...])
    z = lax.dot_general(y, w2_ref[...], (((1,), (1,)), ((), ())),
                        preferred_element_type=jnp.float32)
    z = z + b2_ref[...]
    z_ref[...] = jnp.where(m_ref[...], z, -jnp.inf)


def kernel(x, edge_index, mask, W1, att_src1, att_dst1, b1,
           W2, att_src2, att_dst2, b2, fc1_w, fc1_b, fc2_w, fc2_b):
    loop = jnp.arange(N, dtype=jnp.int32)

    def _slabs(e, geo):
        chunk, nch_a, nch_b = geo
        off_b = NS * nch_a * chunk
        total = NS * (nch_a + nch_b) * chunk
        flat = jnp.concatenate(
            [e.astype(jnp.int32), loop,
             jnp.zeros((total - ETOT,), jnp.int32)])
        a = flat[:off_b].reshape(NS, nch_a, chunk)
        bpart = flat[off_b:].reshape(NS, nch_b, chunk)
        padch = jnp.zeros((NS, nch_a - nch_b, chunk), jnp.int32)
        return jnp.concatenate([a, jnp.concatenate([bpart, padch], axis=1)])

    srcp1 = _slabs(edge_index[0], GEO1)
    dstp1 = _slabs(edge_index[1], GEO1)
    srcp2 = srcp1 if GEO2 == GEO1 else _slabs(edge_index[0], GEO2)
    dstp2 = dstp1 if GEO2 == GEO1 else _slabs(edge_index[1], GEO2)
    z1 = jnp.zeros((NPAD, CP1), jnp.float32)
    z2 = jnp.zeros((NPAD, CP2), jnp.float32)

    h1pad, aa1 = pl.pallas_call(
        _tc_prep_body,
        out_shape=(jax.ShapeDtypeStruct((N, CP1), jnp.float32),
                   jax.ShapeDtypeStruct((2, N), jnp.float32)),
    )(x, W1, att_src1.reshape(1, C1), att_dst1.reshape(1, C1))

    p1 = _edge_pass(CP1, GEO1)(h1pad, srcp1, dstp1, aa1, z1)

    h2pad, aa2 = pl.pallas_call(
        _tc_mid_body,
        out_shape=(jax.ShapeDtypeStruct((N, CP2), jnp.float32),
                   jax.ShapeDtypeStruct((2, N), jnp.float32)),
    )(p1, b1.reshape(1, C1), W2, att_src2.reshape(1, C2),
      att_dst2.reshape(1, C2))

    p2 = _edge_pass(CP2, GEO2)(h2pad, srcp2, dstp2, aa2, z2)

    o2 = pl.pallas_call(
        _tc_fina_body,
        out_shape=jax.ShapeDtypeStruct((N, C2), jnp.float32),
    )(p2, b2.reshape(1, C2))

    o2f = o2.reshape(B, NODES_PER_G * C2)

    out = pl.pallas_call(
        _tc_finb_body,
        out_shape=jax.ShapeDtypeStruct((B, OUT_DIM), jnp.float32),
    )(o2f, fc1_w, fc1_b.reshape(1, OUT_DIM), fc2_w,
      fc2_b.reshape(1, OUT_DIM), mask)
    return out
